# baseline (speedup 1.0000x reference)
import functools

import jax
import jax.numpy as jnp
from jax import lax
from jax.experimental import pallas as pl
from jax.experimental.pallas import tpu as pltpu
from jax.experimental.pallas import tpu_sc as plsc

ROW_BLK = 2000
EDGE_CHUNK = 200
NODE_CHUNK = 160
NS = 16
H_HALF = 16


def _proj_body(x_ref, wl_ref, bl_ref, wg_ref, za_ref, zb_ref):
    z1 = jnp.dot(x_ref[...], wl_ref[...], preferred_element_type=jnp.float32)
    z1 = z1 + bl_ref[...]
    z2 = jnp.dot(z1, wg_ref[...], preferred_element_type=jnp.float32)
    za_ref[...] = z2[:, :H_HALF]
    zb_ref[...] = z2[:, H_HALF:]


def _project(x, W_lin, b_lin, W_gc):
    n, d = x.shape
    h1 = W_lin.shape[1]
    h2 = W_gc.shape[1]
    grid = n // ROW_BLK
    return pl.pallas_call(
        _proj_body,
        grid=(grid,),
        in_specs=[
            pl.BlockSpec((ROW_BLK, d), lambda i: (i, 0)),
            pl.BlockSpec((d, h1), lambda i: (0, 0)),
            pl.BlockSpec((1, h1), lambda i: (0, 0)),
            pl.BlockSpec((h1, h2), lambda i: (0, 0)),
        ],
        out_specs=[
            pl.BlockSpec((ROW_BLK, H_HALF), lambda i: (i, 0)),
            pl.BlockSpec((ROW_BLK, H_HALF), lambda i: (i, 0)),
        ],
        out_shape=[
            jax.ShapeDtypeStruct((n, H_HALF), jnp.float32),
            jax.ShapeDtypeStruct((n, H_HALF), jnp.float32),
        ],
    )(x, W_lin, b_lin.reshape(1, h1), W_gc)


def _sc_aggregate(z2a, z2b, edge_index, b_gc):
    n = z2a.shape[0]
    e = edge_index.shape[1]
    h2 = b_gc.shape[0]
    ept = e // NS
    nchunks = ept // EDGE_CHUNK
    node_chunks = n // NODE_CHUNK
    ncpt = node_chunks // NS
    ncrem = node_chunks - ncpt * NS
    mesh = plsc.VectorSubcoreMesh(core_axis_name="c", subcore_axis_name="s")

    @functools.partial(
        pl.kernel,
        out_type=jax.ShapeDtypeStruct((n, h2), jnp.float32),
        mesh=mesh,
        compiler_params=pltpu.CompilerParams(use_tc_tiling_on_sc=False),
        scratch_types=[
            pltpu.VMEM_SHARED((n, H_HALF), jnp.float32),
            pltpu.VMEM_SHARED((n,), jnp.float32),
            pltpu.VMEM((EDGE_CHUNK,), jnp.int32),
            pltpu.VMEM((EDGE_CHUNK,), jnp.int32),
            pltpu.VMEM((EDGE_CHUNK,), jnp.int32),
            pltpu.VMEM((EDGE_CHUNK,), jnp.int32),
            pltpu.VMEM((EDGE_CHUNK,), jnp.int32),
            pltpu.VMEM((EDGE_CHUNK,), jnp.int32),
            pltpu.VMEM((EDGE_CHUNK,), jnp.int32),
            pltpu.VMEM((EDGE_CHUNK,), jnp.int32),
            pltpu.VMEM((EDGE_CHUNK, H_HALF), jnp.float32),
            pltpu.VMEM((EDGE_CHUNK, H_HALF), jnp.float32),
            pltpu.VMEM((EDGE_CHUNK,), jnp.float32),
            pltpu.VMEM((NODE_CHUNK,), jnp.float32),
            pltpu.VMEM((NODE_CHUNK,), jnp.float32),
            pltpu.VMEM((32,), jnp.float32),
            pltpu.SemaphoreType.DMA,
            pltpu.SemaphoreType.DMA,
            pltpu.SemaphoreType.DMA,
            pltpu.SemaphoreType.DMA,
            pltpu.SemaphoreType.DMA,
            pltpu.SemaphoreType.DMA,
            pltpu.SemaphoreType.DMA,
        ],
    )
    def body(za_hbm, zb_hbm, ei_hbm, bgc_hbm, out_hbm,
             agg_sh, deg_sh, src0, src1, src2, src3, dst0, dst1, dst2, dst3,
             gbufA, gbufB, onesb, degb, recb, bgcb,
             semi0, semi1, semi2, semi3, semg, semsc0, semsc1):
        srcs = (src0, src1, src2, src3)
        dsts = (dst0, dst1, dst2, dst3)
        gbufs = (gbufA, gbufB)
        semis = (semi0, semi1, semi2, semi3)
        semscs = (semsc0, semsc1)
        gbuf = gbufA
        rowsb_dma = gbuf.at[pl.ds(0, NODE_CHUNK)]
        c = lax.axis_index("c")
        s = lax.axis_index("s")

        def fill_ones(i, carry):
            onesb[pl.ds(i * 16, 16)] = jnp.full((16,), 1.0, jnp.float32)
            return carry
        lax.fori_loop(0, EDGE_CHUNK // 16, fill_ones, 0)
        if EDGE_CHUNK % 16:
            onesb[pl.ds(EDGE_CHUNK - 16, 16)] = jnp.full((16,), 1.0, jnp.float32)

        def zero_deg(i, carry):
            degb[pl.ds(i * 16, 16)] = jnp.zeros((16,), jnp.float32)
            return carry
        lax.fori_loop(0, NODE_CHUNK // 16, zero_deg, 0)

        def zero_rows(i, carry):
            gbuf[i] = jnp.zeros((H_HALF,), jnp.float32)
            return carry
        lax.fori_loop(0, NODE_CHUNK, zero_rows, 0)

        pltpu.sync_copy(bgc_hbm, bgcb)

        for j in range(ncpt):
            k = s + NS * j
            pltpu.sync_copy(rowsb_dma, agg_sh.at[pl.ds(k * NODE_CHUNK, NODE_CHUNK)])
            pltpu.sync_copy(degb, deg_sh.at[pl.ds(k * NODE_CHUNK, NODE_CHUNK)])

        @pl.when(s < ncrem)
        def _zero_rem():
            k = ncpt * NS + s
            pltpu.sync_copy(rowsb_dma, agg_sh.at[pl.ds(k * NODE_CHUNK, NODE_CHUNK)])
            pltpu.sync_copy(degb, deg_sh.at[pl.ds(k * NODE_CHUNK, NODE_CHUNK)])

        plsc.subcore_barrier()

        tile_base = s * ept

        def make_loop(table_hbm):
            def issue_idx(k, slot):
                eb = tile_base + k * EDGE_CHUNK
                pltpu.async_copy(ei_hbm.at[0, pl.ds(eb, EDGE_CHUNK)],
                                 srcs[slot], semis[slot])
                pltpu.async_copy(ei_hbm.at[1, pl.ds(eb, EDGE_CHUNK)],
                                 dsts[slot], semis[slot])

            def drain_idx(slot):
                pltpu.make_async_copy(ei_hbm.at[0, pl.ds(0, EDGE_CHUNK)],
                                      srcs[slot], semis[slot]).wait()
                pltpu.make_async_copy(ei_hbm.at[1, pl.ds(0, EDGE_CHUNK)],
                                      dsts[slot], semis[slot]).wait()

            def drain_scatter(b2, slot):
                pltpu.make_async_copy(gbufs[b2], agg_sh.at[dsts[slot]],
                                      semscs[b2]).wait()
                pltpu.make_async_copy(onesb, deg_sh.at[dsts[slot]],
                                      semscs[b2]).wait()

            issue_idx(0, 0)
            issue_idx(1, 1)

            def group(g, carry):
                for u in range(4):
                    k = g * 4 + u
                    b2 = u % 2
                    slot2 = (u + 2) % 4
                    if u >= 2:
                        drain_scatter(b2, slot2)
                    else:
                        @pl.when(g > 0)
                        def _drain():
                            drain_scatter(b2, slot2)
                    @pl.when(k + 2 < nchunks)
                    def _prefetch():
                        issue_idx(k + 2, slot2)

                    drain_idx(u)
                    pltpu.async_copy(table_hbm.at[srcs[u]], gbufs[b2],
                                     semg).wait()
                    pltpu.async_copy(gbufs[b2], agg_sh.at[dsts[u]],
                                     semscs[b2], add=True)
                    pltpu.async_copy(onesb, deg_sh.at[dsts[u]],
                                     semscs[b2], add=True)
                return carry

            lax.fori_loop(0, nchunks // 4, group, 0)
            drain_scatter(0, 2)
            drain_scatter(1, 3)

        @pl.when(c == 0)
        def _loop_a():
            make_loop(za_hbm)

        @pl.when(c == 1)
        def _loop_b():
            make_loop(zb_hbm)

        plsc.subcore_barrier()

        bias_a = bgcb[pl.ds(0, H_HALF)]
        bias_b = bgcb[pl.ds(H_HALF, H_HALF)]

        def norm_chunk(k):
            r0 = k * NODE_CHUNK
            pltpu.sync_copy(agg_sh.at[pl.ds(r0, NODE_CHUNK)], rowsb_dma)
            pltpu.sync_copy(deg_sh.at[pl.ds(r0, NODE_CHUNK)], degb)

            def recips(i, carry):
                d16 = degb[pl.ds(i * 16, 16)]
                recb[pl.ds(i * 16, 16)] = 1.0 / jnp.maximum(d16, 1.0)
                return carry
            lax.fori_loop(0, NODE_CHUNK // 16, recips, 0)

            def norm_rows(bias):
                def fn(i, carry):
                    rec16 = recb[pl.ds(i * 16, 16)]
                    base = i * 16
                    for j in range(16):
                        gbuf[base + j] = jnp.maximum(
                            gbuf[base + j] * rec16[j] + bias, 0.0)
                    return carry
                return fn

            @pl.when(c == 0)
            def _():
                lax.fori_loop(0, NODE_CHUNK // 16, norm_rows(bias_a), 0)
                pltpu.sync_copy(rowsb_dma, out_hbm.at[pl.ds(r0, NODE_CHUNK), pl.ds(0, H_HALF)])

            @pl.when(c == 1)
            def _():
                lax.fori_loop(0, NODE_CHUNK // 16, norm_rows(bias_b), 0)
                pltpu.sync_copy(rowsb_dma, out_hbm.at[pl.ds(r0, NODE_CHUNK), pl.ds(H_HALF, H_HALF)])

        for j in range(ncpt):
            norm_chunk(s + NS * j)

        @pl.when(s < ncrem)
        def _norm_rem():
            norm_chunk(ncpt * NS + s)

    return body(z2a, z2b, edge_index, b_gc)


def kernel(x, edge_index, W_lin, b_lin, W_gc, b_gc):
    ei = edge_index.astype(jnp.int32)
    z2a, z2b = _project(x, W_lin, b_lin, W_gc)
    return _sc_aggregate(z2a, z2b, ei, b_gc)


# speedup vs baseline: 1.2622x; 1.2622x over previous
import functools

import jax
import jax.numpy as jnp
from jax import lax
from jax.experimental import pallas as pl
from jax.experimental.pallas import tpu as pltpu
from jax.experimental.pallas import tpu_sc as plsc

ROW_BLK = 2000
EDGE_CHUNK = 160
NODE_CHUNK = 160
NS = 16
H_HALF = 16
NRING = 5


def _proj_body(x_ref, wl_ref, bl_ref, wg_ref, za_ref, zb_ref):
    z1 = jnp.dot(x_ref[...], wl_ref[...], preferred_element_type=jnp.float32)
    z1 = z1 + bl_ref[...]
    z2 = jnp.dot(z1, wg_ref[...], preferred_element_type=jnp.float32)
    za_ref[...] = z2[:, :H_HALF]
    zb_ref[...] = z2[:, H_HALF:]


def _project(x, W_lin, b_lin, W_gc):
    n, d = x.shape
    h1 = W_lin.shape[1]
    h2 = W_gc.shape[1]
    grid = n // ROW_BLK
    return pl.pallas_call(
        _proj_body,
        grid=(grid,),
        in_specs=[
            pl.BlockSpec((ROW_BLK, d), lambda i: (i, 0)),
            pl.BlockSpec((d, h1), lambda i: (0, 0)),
            pl.BlockSpec((1, h1), lambda i: (0, 0)),
            pl.BlockSpec((h1, h2), lambda i: (0, 0)),
        ],
        out_specs=[
            pl.BlockSpec((ROW_BLK, H_HALF), lambda i: (i, 0)),
            pl.BlockSpec((ROW_BLK, H_HALF), lambda i: (i, 0)),
        ],
        out_shape=[
            jax.ShapeDtypeStruct((n, H_HALF), jnp.float32),
            jax.ShapeDtypeStruct((n, H_HALF), jnp.float32),
        ],
    )(x, W_lin, b_lin.reshape(1, h1), W_gc)


def _sc_aggregate(z2a, z2b, edge_index, b_gc):
    n = z2a.shape[0]
    e = edge_index.shape[1]
    h2 = b_gc.shape[0]
    ept = e // NS
    nchunks = ept // EDGE_CHUNK
    node_chunks = n // NODE_CHUNK
    ncpt = node_chunks // NS
    ncrem = node_chunks - ncpt * NS
    mesh = plsc.VectorSubcoreMesh(core_axis_name="c", subcore_axis_name="s")

    @functools.partial(
        pl.kernel,
        out_type=jax.ShapeDtypeStruct((n, h2), jnp.float32),
        mesh=mesh,
        compiler_params=pltpu.CompilerParams(use_tc_tiling_on_sc=False),
        scratch_types=(
            [
                pltpu.VMEM_SHARED((n, H_HALF), jnp.float32),
                pltpu.VMEM_SHARED((n,), jnp.float32),
            ]
            + [pltpu.VMEM((EDGE_CHUNK,), jnp.int32)] * NRING
            + [pltpu.VMEM((EDGE_CHUNK,), jnp.int32)] * NRING
            + [pltpu.VMEM((EDGE_CHUNK, H_HALF), jnp.float32)] * NRING
            + [
                pltpu.VMEM((EDGE_CHUNK,), jnp.float32),
                pltpu.VMEM((NODE_CHUNK,), jnp.float32),
                pltpu.VMEM((NODE_CHUNK,), jnp.float32),
                pltpu.VMEM((32,), jnp.float32),
            ]
            + [pltpu.SemaphoreType.DMA] * (3 * NRING)
        ),
    )
    def body(za_hbm, zb_hbm, ei_hbm, bgc_hbm, out_hbm, agg_sh, deg_sh, *rest):
        srcs = rest[0:NRING]
        dsts = rest[NRING:2 * NRING]
        gbufs = rest[2 * NRING:3 * NRING]
        onesb, degb, recb, bgcb = rest[3 * NRING:3 * NRING + 4]
        sems = rest[3 * NRING + 4:]
        semis = sems[0:NRING]
        semgs = sems[NRING:2 * NRING]
        semscs = sems[2 * NRING:3 * NRING]
        gbuf = gbufs[0]
        rowsb_dma = gbuf
        c = lax.axis_index("c")
        s = lax.axis_index("s")

        def fill_ones(i, carry):
            onesb[pl.ds(i * 16, 16)] = jnp.full((16,), 1.0, jnp.float32)
            return carry
        lax.fori_loop(0, EDGE_CHUNK // 16, fill_ones, 0)
        if EDGE_CHUNK % 16:
            onesb[pl.ds(EDGE_CHUNK - 16, 16)] = jnp.full((16,), 1.0, jnp.float32)

        def zero_deg(i, carry):
            degb[pl.ds(i * 16, 16)] = jnp.zeros((16,), jnp.float32)
            return carry
        lax.fori_loop(0, NODE_CHUNK // 16, zero_deg, 0)

        def zero_rows(i, carry):
            gbuf[i] = jnp.zeros((H_HALF,), jnp.float32)
            return carry
        lax.fori_loop(0, NODE_CHUNK, zero_rows, 0)

        pltpu.sync_copy(bgc_hbm, bgcb)

        for j in range(ncpt):
            k = s + NS * j
            pltpu.sync_copy(rowsb_dma, agg_sh.at[pl.ds(k * NODE_CHUNK, NODE_CHUNK)])
            pltpu.sync_copy(degb, deg_sh.at[pl.ds(k * NODE_CHUNK, NODE_CHUNK)])

        @pl.when(s < ncrem)
        def _zero_rem():
            k = ncpt * NS + s
            pltpu.sync_copy(rowsb_dma, agg_sh.at[pl.ds(k * NODE_CHUNK, NODE_CHUNK)])
            pltpu.sync_copy(degb, deg_sh.at[pl.ds(k * NODE_CHUNK, NODE_CHUNK)])

        plsc.subcore_barrier()

        tile_base = s * ept

        def make_loop(table_hbm):
            def issue_idx(k, slot):
                eb = tile_base + k * EDGE_CHUNK
                pltpu.async_copy(ei_hbm.at[0, pl.ds(eb, EDGE_CHUNK)],
                                 srcs[slot], semis[slot])
                pltpu.async_copy(ei_hbm.at[1, pl.ds(eb, EDGE_CHUNK)],
                                 dsts[slot], semis[slot])

            def drain_idx(slot):
                pltpu.make_async_copy(ei_hbm.at[0, pl.ds(0, EDGE_CHUNK)],
                                      srcs[slot], semis[slot]).wait()
                pltpu.make_async_copy(ei_hbm.at[1, pl.ds(0, EDGE_CHUNK)],
                                      dsts[slot], semis[slot]).wait()

            def issue_gather(slot):
                pltpu.async_copy(table_hbm.at[srcs[slot]], gbufs[slot],
                                 semgs[slot])

            def drain_gather(slot):
                pltpu.make_async_copy(table_hbm.at[srcs[slot]], gbufs[slot],
                                      semgs[slot]).wait()

            def issue_scatter(slot):
                pltpu.async_copy(gbufs[slot], agg_sh.at[dsts[slot]],
                                 semscs[slot], add=True)
                pltpu.async_copy(onesb, deg_sh.at[dsts[slot]],
                                 semscs[slot], add=True)

            def drain_scatter(slot):
                pltpu.make_async_copy(gbufs[slot], agg_sh.at[dsts[slot]],
                                      semscs[slot]).wait()
                pltpu.make_async_copy(onesb, deg_sh.at[dsts[slot]],
                                      semscs[slot]).wait()

            issue_idx(0, 0)
            issue_idx(1, 1)
            issue_idx(2, 2)
            drain_idx(0)
            issue_gather(0)
            drain_idx(1)
            issue_gather(1)

            def group(g, carry):
                for u in range(NRING):
                    k = g * NRING + u
                    s3 = (u + 3) % NRING
                    s2 = (u + 2) % NRING
                    drain_gather(u)
                    issue_scatter(u)
                    if u >= 2:
                        drain_scatter(s3)
                    else:
                        @pl.when(g > 0)
                        def _drain():
                            drain_scatter(s3)
                    @pl.when(k + 3 < nchunks)
                    def _prefetch():
                        issue_idx(k + 3, s3)

                    @pl.when(k + 2 < nchunks)
                    def _gather_ahead():
                        drain_idx(s2)
                        issue_gather(s2)
                return carry

            lax.fori_loop(0, nchunks // NRING, group, 0)
            drain_scatter((nchunks - 2) % NRING)
            drain_scatter((nchunks - 1) % NRING)

        @pl.when(c == 0)
        def _loop_a():
            make_loop(za_hbm)

        @pl.when(c == 1)
        def _loop_b():
            make_loop(zb_hbm)

        plsc.subcore_barrier()

        bias_a = bgcb[pl.ds(0, H_HALF)]
        bias_b = bgcb[pl.ds(H_HALF, H_HALF)]

        def norm_chunk(k):
            r0 = k * NODE_CHUNK
            pltpu.sync_copy(agg_sh.at[pl.ds(r0, NODE_CHUNK)], rowsb_dma)
            pltpu.sync_copy(deg_sh.at[pl.ds(r0, NODE_CHUNK)], degb)

            def recips(i, carry):
                d16 = degb[pl.ds(i * 16, 16)]
                recb[pl.ds(i * 16, 16)] = 1.0 / jnp.maximum(d16, 1.0)
                return carry
            lax.fori_loop(0, NODE_CHUNK // 16, recips, 0)

            def norm_rows(bias):
                def fn(i, carry):
                    rec16 = recb[pl.ds(i * 16, 16)]
                    base = i * 16
                    for j in range(16):
                        gbuf[base + j] = jnp.maximum(
                            gbuf[base + j] * rec16[j] + bias, 0.0)
                    return carry
                return fn

            @pl.when(c == 0)
            def _():
                lax.fori_loop(0, NODE_CHUNK // 16, norm_rows(bias_a), 0)
                pltpu.sync_copy(rowsb_dma, out_hbm.at[pl.ds(r0, NODE_CHUNK), pl.ds(0, H_HALF)])

            @pl.when(c == 1)
            def _():
                lax.fori_loop(0, NODE_CHUNK // 16, norm_rows(bias_b), 0)
                pltpu.sync_copy(rowsb_dma, out_hbm.at[pl.ds(r0, NODE_CHUNK), pl.ds(H_HALF, H_HALF)])

        for j in range(ncpt):
            norm_chunk(s + NS * j)

        @pl.when(s < ncrem)
        def _norm_rem():
            norm_chunk(ncpt * NS + s)

    return body(z2a, z2b, edge_index, b_gc)


def kernel(x, edge_index, W_lin, b_lin, W_gc, b_gc):
    ei = edge_index.astype(jnp.int32)
    z2a, z2b = _project(x, W_lin, b_lin, W_gc)
    return _sc_aggregate(z2a, z2b, ei, b_gc)


# speedup vs baseline: 1.3641x; 1.0807x over previous
import functools

import jax
import jax.numpy as jnp
from jax import lax
from jax.experimental import pallas as pl
from jax.experimental.pallas import tpu as pltpu
from jax.experimental.pallas import tpu_sc as plsc

ROW_BLK = 2000
EDGE_CHUNK = 200
NODE_CHUNK = 160
NS = 16
H_HALF = 16
NRING = 5


def _proj_body(x_ref, wl_ref, bl_ref, wg_ref, za_ref, zb_ref):
    z1 = jnp.dot(x_ref[...], wl_ref[...], preferred_element_type=jnp.float32)
    z1 = z1 + bl_ref[...]
    z2 = jnp.dot(z1, wg_ref[...], preferred_element_type=jnp.float32)
    za_ref[...] = z2[:, :H_HALF]
    zb_ref[...] = z2[:, H_HALF:]


def _project(x, W_lin, b_lin, W_gc):
    n, d = x.shape
    h1 = W_lin.shape[1]
    h2 = W_gc.shape[1]
    grid = n // ROW_BLK
    return pl.pallas_call(
        _proj_body,
        grid=(grid,),
        in_specs=[
            pl.BlockSpec((ROW_BLK, d), lambda i: (i, 0)),
            pl.BlockSpec((d, h1), lambda i: (0, 0)),
            pl.BlockSpec((1, h1), lambda i: (0, 0)),
            pl.BlockSpec((h1, h2), lambda i: (0, 0)),
        ],
        out_specs=[
            pl.BlockSpec((ROW_BLK, H_HALF), lambda i: (i, 0)),
            pl.BlockSpec((ROW_BLK, H_HALF), lambda i: (i, 0)),
        ],
        out_shape=[
            jax.ShapeDtypeStruct((n, H_HALF), jnp.float32),
            jax.ShapeDtypeStruct((n, H_HALF), jnp.float32),
        ],
    )(x, W_lin, b_lin.reshape(1, h1), W_gc)


def _sc_aggregate(z2a, z2b, edge_index, b_gc):
    n = z2a.shape[0]
    e = edge_index.shape[1]
    h2 = b_gc.shape[0]
    ept = e // NS
    nchunks = ept // EDGE_CHUNK
    node_chunks = n // NODE_CHUNK
    ncpt = node_chunks // NS
    ncrem = node_chunks - ncpt * NS
    mesh = plsc.VectorSubcoreMesh(core_axis_name="c", subcore_axis_name="s")

    @functools.partial(
        pl.kernel,
        out_type=jax.ShapeDtypeStruct((n, h2), jnp.float32),
        mesh=mesh,
        compiler_params=pltpu.CompilerParams(use_tc_tiling_on_sc=False),
        scratch_types=(
            [
                pltpu.VMEM_SHARED((n, H_HALF), jnp.float32),
                pltpu.VMEM_SHARED((n,), jnp.float32),
            ]
            + [pltpu.VMEM((EDGE_CHUNK,), jnp.int32)] * NRING
            + [pltpu.VMEM((EDGE_CHUNK,), jnp.int32)] * NRING
            + [pltpu.VMEM((EDGE_CHUNK, H_HALF), jnp.float32)] * NRING
            + [
                pltpu.VMEM((EDGE_CHUNK,), jnp.float32),
                pltpu.VMEM((NODE_CHUNK,), jnp.float32),
                pltpu.VMEM((NODE_CHUNK,), jnp.float32),
                pltpu.VMEM((32,), jnp.float32),
            ]
            + [pltpu.SemaphoreType.DMA] * (3 * NRING)
        ),
    )
    def body(za_hbm, zb_hbm, ei_hbm, bgc_hbm, out_hbm, agg_sh, deg_sh, *rest):
        srcs = rest[0:NRING]
        dsts = rest[NRING:2 * NRING]
        gbufs = rest[2 * NRING:3 * NRING]
        onesb, degb, recb, bgcb = rest[3 * NRING:3 * NRING + 4]
        sems = rest[3 * NRING + 4:]
        semis = sems[0:NRING]
        semgs = sems[NRING:2 * NRING]
        semscs = sems[2 * NRING:3 * NRING]
        gbuf = gbufs[0]
        rowsb_dma = gbuf.at[pl.ds(0, NODE_CHUNK)]
        c = lax.axis_index("c")
        s = lax.axis_index("s")

        def fill_ones(i, carry):
            onesb[pl.ds(i * 16, 16)] = jnp.full((16,), 1.0, jnp.float32)
            return carry
        lax.fori_loop(0, EDGE_CHUNK // 16, fill_ones, 0)
        if EDGE_CHUNK % 16:
            onesb[pl.ds(EDGE_CHUNK - 16, 16)] = jnp.full((16,), 1.0, jnp.float32)

        def zero_deg(i, carry):
            degb[pl.ds(i * 16, 16)] = jnp.zeros((16,), jnp.float32)
            return carry
        lax.fori_loop(0, NODE_CHUNK // 16, zero_deg, 0)

        def zero_rows(i, carry):
            gbuf[i] = jnp.zeros((H_HALF,), jnp.float32)
            return carry
        lax.fori_loop(0, NODE_CHUNK, zero_rows, 0)

        pltpu.sync_copy(bgc_hbm, bgcb)

        for j in range(ncpt):
            k = s + NS * j
            pltpu.sync_copy(rowsb_dma, agg_sh.at[pl.ds(k * NODE_CHUNK, NODE_CHUNK)])
            pltpu.sync_copy(degb, deg_sh.at[pl.ds(k * NODE_CHUNK, NODE_CHUNK)])

        @pl.when(s < ncrem)
        def _zero_rem():
            k = ncpt * NS + s
            pltpu.sync_copy(rowsb_dma, agg_sh.at[pl.ds(k * NODE_CHUNK, NODE_CHUNK)])
            pltpu.sync_copy(degb, deg_sh.at[pl.ds(k * NODE_CHUNK, NODE_CHUNK)])

        plsc.subcore_barrier()

        tile_base = s * ept

        def make_loop(table_hbm):
            def issue_idx(k, slot):
                eb = tile_base + k * EDGE_CHUNK
                pltpu.async_copy(ei_hbm.at[0, pl.ds(eb, EDGE_CHUNK)],
                                 srcs[slot], semis[slot])
                pltpu.async_copy(ei_hbm.at[1, pl.ds(eb, EDGE_CHUNK)],
                                 dsts[slot], semis[slot])

            def drain_idx(slot):
                pltpu.make_async_copy(ei_hbm.at[0, pl.ds(0, EDGE_CHUNK)],
                                      srcs[slot], semis[slot]).wait()
                pltpu.make_async_copy(ei_hbm.at[1, pl.ds(0, EDGE_CHUNK)],
                                      dsts[slot], semis[slot]).wait()

            def issue_gather(slot):
                pltpu.async_copy(table_hbm.at[srcs[slot]], gbufs[slot],
                                 semgs[slot])

            def drain_gather(slot):
                pltpu.make_async_copy(table_hbm.at[srcs[slot]], gbufs[slot],
                                      semgs[slot]).wait()

            def issue_scatter(slot):
                pltpu.async_copy(gbufs[slot], agg_sh.at[dsts[slot]],
                                 semscs[slot], add=True)
                pltpu.async_copy(onesb, deg_sh.at[dsts[slot]],
                                 semscs[slot], add=True)

            def drain_scatter(slot):
                pltpu.make_async_copy(gbufs[slot], agg_sh.at[dsts[slot]],
                                      semscs[slot]).wait()
                pltpu.make_async_copy(onesb, deg_sh.at[dsts[slot]],
                                      semscs[slot]).wait()

            issue_idx(0, 0)
            issue_idx(1, 1)
            issue_idx(2, 2)
            drain_idx(0)
            issue_gather(0)
            drain_idx(1)
            issue_gather(1)

            def group(g, carry):
                for u in range(NRING):
                    k = g * NRING + u
                    s3 = (u + 3) % NRING
                    s2 = (u + 2) % NRING
                    drain_gather(u)
                    issue_scatter(u)
                    if u >= 2:
                        drain_scatter(s3)
                    else:
                        @pl.when(g > 0)
                        def _drain():
                            drain_scatter(s3)
                    @pl.when(k + 3 < nchunks)
                    def _prefetch():
                        issue_idx(k + 3, s3)

                    @pl.when(k + 2 < nchunks)
                    def _gather_ahead():
                        drain_idx(s2)
                        issue_gather(s2)
                return carry

            lax.fori_loop(0, nchunks // NRING, group, 0)
            drain_scatter((nchunks - 2) % NRING)
            drain_scatter((nchunks - 1) % NRING)

        @pl.when(c == 0)
        def _loop_a():
            make_loop(za_hbm)

        @pl.when(c == 1)
        def _loop_b():
            make_loop(zb_hbm)

        plsc.subcore_barrier()

        bias_a = bgcb[pl.ds(0, H_HALF)]
        bias_b = bgcb[pl.ds(H_HALF, H_HALF)]

        def norm_chunk(k):
            r0 = k * NODE_CHUNK
            pltpu.sync_copy(agg_sh.at[pl.ds(r0, NODE_CHUNK)], rowsb_dma)
            pltpu.sync_copy(deg_sh.at[pl.ds(r0, NODE_CHUNK)], degb)

            def recips(i, carry):
                d16 = degb[pl.ds(i * 16, 16)]
                recb[pl.ds(i * 16, 16)] = 1.0 / jnp.maximum(d16, 1.0)
                return carry
            lax.fori_loop(0, NODE_CHUNK // 16, recips, 0)

            def norm_rows(bias):
                def fn(i, carry):
                    rec16 = recb[pl.ds(i * 16, 16)]
                    base = i * 16
                    for j in range(16):
                        gbuf[base + j] = jnp.maximum(
                            gbuf[base + j] * rec16[j] + bias, 0.0)
                    return carry
                return fn

            @pl.when(c == 0)
            def _():
                lax.fori_loop(0, NODE_CHUNK // 16, norm_rows(bias_a), 0)
                pltpu.sync_copy(rowsb_dma, out_hbm.at[pl.ds(r0, NODE_CHUNK), pl.ds(0, H_HALF)])

            @pl.when(c == 1)
            def _():
                lax.fori_loop(0, NODE_CHUNK // 16, norm_rows(bias_b), 0)
                pltpu.sync_copy(rowsb_dma, out_hbm.at[pl.ds(r0, NODE_CHUNK), pl.ds(H_HALF, H_HALF)])

        for j in range(ncpt):
            norm_chunk(s + NS * j)

        @pl.when(s < ncrem)
        def _norm_rem():
            norm_chunk(ncpt * NS + s)

    return body(z2a, z2b, edge_index, b_gc)


def kernel(x, edge_index, W_lin, b_lin, W_gc, b_gc):
    ei = edge_index.astype(jnp.int32)
    z2a, z2b = _project(x, W_lin, b_lin, W_gc)
    return _sc_aggregate(z2a, z2b, ei, b_gc)


# speedup vs baseline: 1.4176x; 1.0392x over previous
import functools

import jax
import jax.numpy as jnp
from jax import lax
from jax.experimental import pallas as pl
from jax.experimental.pallas import tpu as pltpu
from jax.experimental.pallas import tpu_sc as plsc

ROW_BLK = 2000
EDGE_CHUNK = 200
NODE_CHUNK = 160
NS = 16
H_HALF = 16
NRING = 5


def _proj_body(x_ref, wl_ref, bl_ref, wg_ref, za_ref, zb_ref):
    z1 = jnp.dot(x_ref[...], wl_ref[...], preferred_element_type=jnp.float32)
    z1 = z1 + bl_ref[...]
    z2 = jnp.dot(z1, wg_ref[...], preferred_element_type=jnp.float32)
    za_ref[...] = z2[:, :H_HALF]
    zb_ref[...] = z2[:, H_HALF:]


def _project(x, W_lin, b_lin, W_gc):
    n, d = x.shape
    h1 = W_lin.shape[1]
    h2 = W_gc.shape[1]
    grid = n // ROW_BLK
    return pl.pallas_call(
        _proj_body,
        grid=(grid,),
        in_specs=[
            pl.BlockSpec((ROW_BLK, d), lambda i: (i, 0)),
            pl.BlockSpec((d, h1), lambda i: (0, 0)),
            pl.BlockSpec((1, h1), lambda i: (0, 0)),
            pl.BlockSpec((h1, h2), lambda i: (0, 0)),
        ],
        out_specs=[
            pl.BlockSpec((ROW_BLK, H_HALF), lambda i: (i, 0)),
            pl.BlockSpec((ROW_BLK, H_HALF), lambda i: (i, 0)),
        ],
        out_shape=[
            jax.ShapeDtypeStruct((n, H_HALF), jnp.float32),
            jax.ShapeDtypeStruct((n, H_HALF), jnp.float32),
        ],
    )(x, W_lin, b_lin.reshape(1, h1), W_gc)


def _sc_aggregate(z2a, z2b, edge_index, b_gc):
    n = z2a.shape[0]
    e = edge_index.shape[1]
    h2 = b_gc.shape[0]
    ept = e // NS
    nchunks = ept // EDGE_CHUNK
    node_chunks = n // NODE_CHUNK
    ncpt = node_chunks // NS
    ncrem = node_chunks - ncpt * NS
    mesh = plsc.VectorSubcoreMesh(core_axis_name="c", subcore_axis_name="s")

    @functools.partial(
        pl.kernel,
        out_type=jax.ShapeDtypeStruct((n, h2), jnp.float32),
        mesh=mesh,
        compiler_params=pltpu.CompilerParams(use_tc_tiling_on_sc=False),
        scratch_types=(
            [
                pltpu.VMEM_SHARED((n, H_HALF), jnp.float32),
                pltpu.VMEM_SHARED((n,), jnp.float32),
            ]
            + [pltpu.VMEM((EDGE_CHUNK,), jnp.int32)] * NRING
            + [pltpu.VMEM((EDGE_CHUNK,), jnp.int32)] * NRING
            + [pltpu.VMEM((EDGE_CHUNK, H_HALF), jnp.float32)] * NRING
            + [pltpu.VMEM((EDGE_CHUNK,), jnp.float32)]
            + [pltpu.VMEM((NODE_CHUNK,), jnp.float32)] * 3
            + [pltpu.VMEM((32,), jnp.float32)]
            + [pltpu.SemaphoreType.DMA] * (3 * NRING)
        ),
    )
    def body(za_hbm, zb_hbm, ei_hbm, bgc_hbm, out_hbm, agg_sh, deg_sh, *rest):
        srcs = rest[0:NRING]
        dsts = rest[NRING:2 * NRING]
        gbufs = rest[2 * NRING:3 * NRING]
        onesb = rest[3 * NRING]
        degs = rest[3 * NRING + 1:3 * NRING + 4]
        degb = degs[0]
        bgcb = rest[3 * NRING + 4]
        sems = rest[3 * NRING + 5:]
        semis = sems[0:NRING]
        semgs = sems[NRING:2 * NRING]
        semscs = sems[2 * NRING:3 * NRING]
        gbuf = gbufs[0]
        rowsb_dma = gbuf.at[pl.ds(0, NODE_CHUNK)]
        c = lax.axis_index("c")
        s = lax.axis_index("s")

        def fill_ones(i, carry):
            onesb[pl.ds(i * 16, 16)] = jnp.full((16,), 1.0, jnp.float32)
            return carry
        lax.fori_loop(0, EDGE_CHUNK // 16, fill_ones, 0)
        if EDGE_CHUNK % 16:
            onesb[pl.ds(EDGE_CHUNK - 16, 16)] = jnp.full((16,), 1.0, jnp.float32)

        def zero_deg(i, carry):
            degb[pl.ds(i * 16, 16)] = jnp.zeros((16,), jnp.float32)
            return carry
        lax.fori_loop(0, NODE_CHUNK // 16, zero_deg, 0)

        def zero_rows(i, carry):
            gbuf[i] = jnp.zeros((H_HALF,), jnp.float32)
            return carry
        lax.fori_loop(0, NODE_CHUNK, zero_rows, 0)

        pltpu.sync_copy(bgc_hbm, bgcb)

        for j in range(ncpt):
            k = s + NS * j
            pltpu.async_copy(rowsb_dma,
                             agg_sh.at[pl.ds(k * NODE_CHUNK, NODE_CHUNK)],
                             semis[0])
            pltpu.async_copy(degb,
                             deg_sh.at[pl.ds(k * NODE_CHUNK, NODE_CHUNK)],
                             semis[1])

        @pl.when(s < ncrem)
        def _zero_rem():
            k = ncpt * NS + s
            pltpu.async_copy(rowsb_dma,
                             agg_sh.at[pl.ds(k * NODE_CHUNK, NODE_CHUNK)],
                             semis[0])
            pltpu.async_copy(degb,
                             deg_sh.at[pl.ds(k * NODE_CHUNK, NODE_CHUNK)],
                             semis[1])

        for j in range(ncpt):
            pltpu.make_async_copy(
                rowsb_dma, agg_sh.at[pl.ds(0, NODE_CHUNK)], semis[0]).wait()
            pltpu.make_async_copy(
                degb, deg_sh.at[pl.ds(0, NODE_CHUNK)], semis[1]).wait()

        @pl.when(s < ncrem)
        def _drain_rem():
            pltpu.make_async_copy(
                rowsb_dma, agg_sh.at[pl.ds(0, NODE_CHUNK)], semis[0]).wait()
            pltpu.make_async_copy(
                degb, deg_sh.at[pl.ds(0, NODE_CHUNK)], semis[1]).wait()

        plsc.subcore_barrier()

        tile_base = s * ept

        def make_loop(table_hbm):
            def issue_idx(k, slot):
                eb = tile_base + k * EDGE_CHUNK
                pltpu.async_copy(ei_hbm.at[0, pl.ds(eb, EDGE_CHUNK)],
                                 srcs[slot], semis[slot])
                pltpu.async_copy(ei_hbm.at[1, pl.ds(eb, EDGE_CHUNK)],
                                 dsts[slot], semis[slot])

            def drain_idx(slot):
                pltpu.make_async_copy(ei_hbm.at[0, pl.ds(0, EDGE_CHUNK)],
                                      srcs[slot], semis[slot]).wait()
                pltpu.make_async_copy(ei_hbm.at[1, pl.ds(0, EDGE_CHUNK)],
                                      dsts[slot], semis[slot]).wait()

            def issue_gather(slot):
                pltpu.async_copy(table_hbm.at[srcs[slot]], gbufs[slot],
                                 semgs[slot])

            def drain_gather(slot):
                pltpu.make_async_copy(table_hbm.at[srcs[slot]], gbufs[slot],
                                      semgs[slot]).wait()

            def issue_scatter(slot):
                pltpu.async_copy(gbufs[slot], agg_sh.at[dsts[slot]],
                                 semscs[slot], add=True)
                pltpu.async_copy(onesb, deg_sh.at[dsts[slot]],
                                 semscs[slot], add=True)

            def drain_scatter(slot):
                pltpu.make_async_copy(gbufs[slot], agg_sh.at[dsts[slot]],
                                      semscs[slot]).wait()
                pltpu.make_async_copy(onesb, deg_sh.at[dsts[slot]],
                                      semscs[slot]).wait()

            issue_idx(0, 0)
            issue_idx(1, 1)
            issue_idx(2, 2)
            drain_idx(0)
            issue_gather(0)
            drain_idx(1)
            issue_gather(1)

            def group(g, carry):
                for u in range(NRING):
                    k = g * NRING + u
                    s3 = (u + 3) % NRING
                    s2 = (u + 2) % NRING
                    drain_gather(u)
                    issue_scatter(u)
                    if u >= 2:
                        drain_scatter(s3)
                    else:
                        @pl.when(g > 0)
                        def _drain():
                            drain_scatter(s3)
                    @pl.when(k + 3 < nchunks)
                    def _prefetch():
                        issue_idx(k + 3, s3)

                    @pl.when(k + 2 < nchunks)
                    def _gather_ahead():
                        drain_idx(s2)
                        issue_gather(s2)
                return carry

            lax.fori_loop(0, nchunks // NRING, group, 0)
            drain_scatter((nchunks - 2) % NRING)
            drain_scatter((nchunks - 1) % NRING)

        @pl.when(c == 0)
        def _loop_a():
            make_loop(za_hbm)

        @pl.when(c == 1)
        def _loop_b():
            make_loop(zb_hbm)

        plsc.subcore_barrier()

        bias_a = bgcb[pl.ds(0, H_HALF)]
        bias_b = bgcb[pl.ds(H_HALF, H_HALF)]
        rbufs = tuple(gbufs[v].at[pl.ds(0, NODE_CHUNK)] for v in range(3))
        col0 = c * H_HALF

        def row0_of(j):
            return (s + NS * j) * NODE_CHUNK

        def issue_in(j, v):
            r0 = row0_of(j)
            pltpu.async_copy(agg_sh.at[pl.ds(r0, NODE_CHUNK)], rbufs[v],
                             semis[v])
            pltpu.async_copy(deg_sh.at[pl.ds(r0, NODE_CHUNK)], degs[v],
                             semis[v])

        def drain_in(v):
            pltpu.make_async_copy(agg_sh.at[pl.ds(0, NODE_CHUNK)], rbufs[v],
                                  semis[v]).wait()
            pltpu.make_async_copy(deg_sh.at[pl.ds(0, NODE_CHUNK)], degs[v],
                                  semis[v]).wait()

        def issue_out(j, v):
            pltpu.async_copy(rbufs[v],
                             out_hbm.at[pl.ds(row0_of(j), NODE_CHUNK),
                                        pl.ds(col0, H_HALF)],
                             semgs[v])

        def drain_out(v):
            pltpu.make_async_copy(rbufs[v],
                                  out_hbm.at[pl.ds(0, NODE_CHUNK),
                                             pl.ds(col0, H_HALF)],
                                  semgs[v]).wait()

        def norm_rows(bias, v):
            rows = gbufs[v]

            def fn(i, carry):
                d16 = degs[v][pl.ds(i * 16, 16)]
                rec16 = 1.0 / jnp.maximum(d16, 1.0)
                base = i * 16
                for j in range(16):
                    rows[base + j] = jnp.maximum(
                        rows[base + j] * rec16[j] + bias, 0.0)
                return carry
            return fn

        def compute(v):
            @pl.when(c == 0)
            def _():
                lax.fori_loop(0, NODE_CHUNK // 16, norm_rows(bias_a, v), 0)

            @pl.when(c == 1)
            def _():
                lax.fori_loop(0, NODE_CHUNK // 16, norm_rows(bias_b, v), 0)

        np3 = ncpt - ncpt % 3

        if np3 >= 3:
            issue_in(0, 0)
            issue_in(1, 1)

            def norm_group(g, carry):
                for v in range(3):
                    j = g * 3 + v
                    v2 = (v + 2) % 3
                    if v >= 1:
                        drain_out(v2)
                    else:
                        @pl.when(g > 0)
                        def _drain():
                            drain_out(v2)

                    @pl.when(j + 2 < np3)
                    def _prefetch():
                        issue_in(j + 2, v2)

                    drain_in(v)
                    compute(v)
                    issue_out(j, v)
                return carry

            lax.fori_loop(0, np3 // 3, norm_group, 0)
            drain_out((np3 - 1) % 3)

        def norm_serial(j):
            issue_in(j, 0)
            drain_in(0)
            compute(0)
            issue_out(j, 0)
            drain_out(0)

        for j in range(np3, ncpt):
            norm_serial(j)

        @pl.when(s < ncrem)
        def _norm_rem():
            r0 = (ncpt * NS + s) * NODE_CHUNK
            pltpu.async_copy(agg_sh.at[pl.ds(r0, NODE_CHUNK)], rbufs[0],
                             semis[0])
            pltpu.async_copy(deg_sh.at[pl.ds(r0, NODE_CHUNK)], degs[0],
                             semis[0])
            drain_in(0)
            compute(0)
            pltpu.async_copy(rbufs[0],
                             out_hbm.at[pl.ds(r0, NODE_CHUNK),
                                        pl.ds(col0, H_HALF)],
                             semgs[0])
            drain_out(0)

    return body(z2a, z2b, edge_index, b_gc)


def kernel(x, edge_index, W_lin, b_lin, W_gc, b_gc):
    ei = edge_index.astype(jnp.int32)
    z2a, z2b = _project(x, W_lin, b_lin, W_gc)
    return _sc_aggregate(z2a, z2b, ei, b_gc)
